# Initial kernel scaffold; baseline (speedup 1.0000x reference)
import jax
import jax.numpy as jnp
from jax.experimental import pallas as pl


def kernel(gt_logits, gen_logits):
    raise NotImplementedError("write your pallas kernel here")


# speedup vs baseline: 1.0506x; 1.0506x over previous
import functools

import jax
import jax.numpy as jnp
from jax import lax
from jax.experimental import pallas as pl
from jax.experimental.pallas import tpu as pltpu
from jax.experimental.pallas import tpu_sc as plsc

N_ELEM = 16384
NT = 16
SHARD = N_ELEM // NT
NV = SHARD // 16
L = 16


def _log2(x):
    return x.bit_length() - 1


def _sigmoid(x):
    e = jnp.exp(-jnp.abs(x))
    return jnp.where(x >= 0, 1.0 / (1.0 + e), e / (1.0 + e))


def _local_sort(buf, scratch):
    def sort_pass(i, _):
        x = buf[pl.ds(i * L, L)]
        sk, _sv = plsc.sort_key_val(x, x)
        buf[pl.ds(i * L, L)] = sk
        return 0

    lax.fori_loop(0, NV, sort_pass, 0)

    m = 16
    while m < SHARD:
        nv = m // L
        npairs = SHARD // (2 * m)

        def rev_body(p, _, nv=nv, m=m):
            base = p * 2 * m + m
            for j in range(nv):
                scratch[pl.ds(j * L, L)] = jnp.flip(
                    buf[pl.ds(base + (nv - 1 - j) * L, L)], 0)
            for j in range(nv):
                buf[pl.ds(base + j * L, L)] = scratch[pl.ds(j * L, L)]
            return 0

        lax.fori_loop(0, npairs, rev_body, 0)

        s = m
        while s >= 16:
            sv = s // L
            shift = _log2(sv)
            mask = sv - 1

            def stage_body(u, _, s=s, shift=shift, mask=mask, sv=sv):
                g = lax.shift_right_logical(u, shift)
                r = lax.bitwise_and(u, mask)
                b1 = (g * (2 * sv) + r) * L
                b2 = b1 + s
                a = buf[pl.ds(b1, L)]
                b = buf[pl.ds(b2, L)]
                buf[pl.ds(b1, L)] = jnp.minimum(a, b)
                buf[pl.ds(b2, L)] = jnp.maximum(a, b)
                return 0

            lax.fori_loop(0, SHARD // (2 * L), stage_body, 0)
            s //= 2

        lax.fori_loop(0, NV, sort_pass, 0)
        m *= 2


def _count_search(akeys, base, q, flag_le):
    lo = jnp.zeros((L,), jnp.int32)
    s = SHARD // 2
    while s >= 1:
        idx = lo + (base + s - 1)
        v = plsc.load_gather(akeys, [idx])
        take = (v < q) | (flag_le & (v == q))
        lo = jnp.where(take, lo + s, lo)
        s //= 2
    return lo


def _tec_body(gt_hbm, gen_hbm, out_hbm,
              lg, keys_c, scratch, akeys, stats_my, stats_all, outv,
              shared_keys, shared_stats):
    w = lax.axis_index("s")
    wvec = jnp.broadcast_to(w, (L,))

    zero = jnp.zeros((L,), jnp.float32)
    stat_acc = [zero] * 6

    for cls, src in ((0, gt_hbm), (1, gen_hbm)):
        pltpu.sync_copy(src.at[pl.ds(w * SHARD, SHARD)], lg)

        def sig_body(i, carry, cls=cls):
            npos, eq, ssum = carry
            x = lg[pl.ds(i * L, L)]
            sv = _sigmoid(x)
            npos = npos + jnp.where(sv > 0.5, 1.0, 0.0)
            eq = eq + jnp.where(sv == 0.5, 1.0, 0.0)
            ssum = ssum + sv
            keys_c[pl.ds(i * L, L)] = sv
            return npos, eq, ssum

        npos, eq, ssum = lax.fori_loop(
            0, NV, sig_body, (zero, zero, zero))
        stat_acc[0 + cls] = npos
        stat_acc[2 + cls] = eq
        stat_acc[4 + cls] = ssum

        _local_sort(keys_c, scratch)
        pltpu.sync_copy(
            keys_c, shared_keys.at[pl.ds(cls * N_ELEM + w * SHARD, SHARD)])

    plsc.subcore_barrier()

    pltpu.sync_copy(shared_keys, akeys)

    iota = lax.iota(jnp.int32, L)

    true_vec = jnp.full((L,), True)

    def search_body(v, carry):
        pr_acc, r_acc = carry
        qb = akeys[pl.ds(w * SHARD + v * L, L)]
        k = iota + v * L

        total = jnp.zeros((L,), jnp.int32)
        sel_own = jnp.zeros((L,), jnp.int32)
        for t in range(NT):
            tvec = jnp.full((L,), t, jnp.int32)
            flag_le = tvec >= wvec
            lo = _count_search(akeys, t * SHARD, qb, flag_le)
            total = total + (SHARD - lo)
            sel_own = sel_own + jnp.where(tvec == wvec, lo, 0)
        cf = jnp.zeros((L,), jnp.int32)
        for t in range(NT):
            lo = _count_search(akeys, N_ELEM + t * SHARD, qb, true_vec)
            cf = cf + (SHARD - lo)

        above = total - (SHARD - sel_own) + (SHARD - 1 - k)
        r = (above + 1).astype(jnp.float32)
        big_r = r + cf.astype(jnp.float32)
        pr_acc = pr_acc + r / (big_r + 1e-08)
        r_acc = r_acc + big_r
        return pr_acc, r_acc

    pr_acc, r_acc = lax.fori_loop(0, NV, search_body, (zero, zero))

    for j, vec in enumerate(stat_acc + [pr_acc, r_acc]):
        stats_my[j] = vec
    pltpu.sync_copy(stats_my, shared_stats.at[w])
    plsc.subcore_barrier()

    @pl.when(w == 0)
    def _():
        pltpu.sync_copy(shared_stats, stats_all)
        tot = []
        for j in range(8):
            acc = jnp.zeros((L,), jnp.float32)
            for t in range(NT):
                acc = acc + stats_all[t, j]
            tot.append(jnp.broadcast_to(jnp.sum(acc), (L,)))
        n_pr, n_pf, eq_r, eq_f, sum_r, sum_f, pr_sum, r_sum = tot

        p = jnp.full((L,), float(N_ELEM), jnp.float32)
        n = jnp.full((L,), float(N_ELEM), jnp.float32)
        accuracy = (n_pr + (n - n_pf)) / (p + n)
        tp = n_pr
        fp = n_pf
        fn = p - n_pr
        precision = tp / (tp + fp + 1e-08)
        recall = tp / (tp + fn + 1e-08)
        f1 = 2.0 * precision * recall / (precision + recall + 1e-08)
        pr_auc = pr_sum / (p + 1e-08)
        sum_pos_ranks = float((2 * N_ELEM + 1) * N_ELEM) - r_sum
        roc_auc = (sum_pos_ranks - float(N_ELEM * (N_ELEM + 1) // 2)) \
            / float(N_ELEM * N_ELEM)
        gap = sum_r / p - sum_f / n
        prob_fake = sum_f / n
        ppf = (n_pr + n_pf) / (p + n)
        acc05 = (eq_r / p + eq_f / n) * 0.5

        vals = [accuracy, precision, recall, f1, pr_auc, roc_auc,
                gap, prob_fake, ppf, acc05]
        ov = jnp.zeros((L,), jnp.float32)
        for i, s in enumerate(vals):
            ov = jnp.where(iota == i, s, ov)
        outv[...] = ov
        pltpu.sync_copy(outv, out_hbm)


@jax.jit
def _run(gt, gen):
    mesh = plsc.VectorSubcoreMesh(
        core_axis_name="c", subcore_axis_name="s", num_cores=1)
    f = pl.kernel(
        _tec_body,
        out_type=jax.ShapeDtypeStruct((L,), jnp.float32),
        mesh=mesh,
        compiler_params=pltpu.CompilerParams(needs_layout_passes=False),
        scratch_types=[
            pltpu.VMEM((SHARD,), jnp.float32),
            pltpu.VMEM((SHARD,), jnp.float32),
            pltpu.VMEM((SHARD // 2,), jnp.float32),
            pltpu.VMEM((2 * N_ELEM,), jnp.float32),
            pltpu.VMEM((8, L), jnp.float32),
            pltpu.VMEM((NT, 8, L), jnp.float32),
            pltpu.VMEM((L,), jnp.float32),
            pltpu.VMEM_SHARED((2 * N_ELEM,), jnp.float32),
            pltpu.VMEM_SHARED((NT, 8, L), jnp.float32),
        ],
    )
    return f(gt, gen)


def kernel(gt_logits, gen_logits):
    gt = jnp.squeeze(gt_logits, -1)
    gen = jnp.squeeze(gen_logits, -1)
    out = _run(gt, gen)
    (accuracy, precision, recall, f1, pr_auc, roc_auc,
     gap, prob_fake, ppf, acc05) = (out[i] for i in range(10))
    return (accuracy, precision, recall, f1, pr_auc, roc_auc,
            gap, prob_fake, ppf, acc05)


# speedup vs baseline: 1.1477x; 1.0924x over previous
import jax
import jax.numpy as jnp
from jax import lax
from jax.experimental import pallas as pl
from jax.experimental.pallas import tpu as pltpu
from jax.experimental.pallas import tpu_sc as plsc

N_ELEM = 16384
NT = 16
CH = 2048
NVC = CH // 16
QB = N_ELEM // NT
L = 16


def _log2(x):
    return x.bit_length() - 1


def _sigmoid(x):
    e = jnp.exp(-jnp.abs(x))
    return jnp.where(x >= 0, 1.0 / (1.0 + e), e / (1.0 + e))


def _sort_pass(buf, n_vregs):
    def body(i, _):
        x = buf[pl.ds(i * L, L)]
        sk, _unused = plsc.sort_key_val(x, x)
        buf[pl.ds(i * L, L)] = sk
        return 0

    lax.fori_loop(0, n_vregs, body, 0)


def _ce_stages(buf, size, s_hi):
    s = s_hi
    while s >= 16:
        sv = s // L
        shift = _log2(sv)
        mask = sv - 1

        def body(u, _, s=s, shift=shift, mask=mask, sv=sv):
            g = lax.shift_right_logical(u, shift)
            r = lax.bitwise_and(u, mask)
            b1 = (g * (2 * sv) + r) * L
            b2 = b1 + s
            a = buf[pl.ds(b1, L)]
            b = buf[pl.ds(b2, L)]
            buf[pl.ds(b1, L)] = jnp.minimum(a, b)
            buf[pl.ds(b2, L)] = jnp.maximum(a, b)
            return 0

        lax.fori_loop(0, size // (2 * L), body, 0)
        s //= 2


def _local_sort(buf, scratch, size):
    nv_all = size // L
    _sort_pass(buf, nv_all)
    m = 16
    while m < size:
        nv = m // L
        npairs = size // (2 * m)

        def rev_body(p, _, nv=nv, m=m):
            base = p * 2 * m + m
            for j in range(nv):
                scratch[pl.ds(j * L, L)] = jnp.flip(
                    buf[pl.ds(base + (nv - 1 - j) * L, L)], 0)
            for j in range(nv):
                buf[pl.ds(base + j * L, L)] = scratch[pl.ds(j * L, L)]
            return 0

        lax.fori_loop(0, npairs, rev_body, 0)
        _ce_stages(buf, size, m)
        _sort_pass(buf, nv_all)
        m *= 2


def _count_le(arr, q, strict=False):
    lo = jnp.zeros((L,), jnp.int32)
    s = N_ELEM // 2
    while s >= 1:
        idx = lo + (s - 1)
        v = plsc.load_gather(arr, [idx])
        take = (v < q) if strict else (v <= q)
        lo = jnp.where(take, lo + s, lo)
        s //= 2
    return lo


def _tec_body(logits_hbm, out_hbm,
              lg, keys_c, pkeys, scratch, sfall, stats_my, stats_all, outv,
              sh_a, sh_b, shared_stats):
    w = lax.axis_index("s")
    base = w * CH

    def cross_mirror(src, dst, k):
        c = lax.bitwise_and(w, 7)
        g = lax.bitwise_and(c, 2 * k - 1)
        part_l = (c - g) + (2 * k - 1 - g)
        part = lax.bitwise_or(lax.bitwise_and(w, 8), part_l)
        is_min = jnp.broadcast_to(g < k, (L,))
        pltpu.sync_copy(src.at[pl.ds(base, CH)], keys_c)
        pltpu.sync_copy(src.at[pl.ds(part * CH, CH)], pkeys)

        def body(j, _):
            a = keys_c[pl.ds(j * L, L)]
            b = jnp.flip(pkeys[pl.ds((NVC - 1) * L - j * L, L)], 0)
            keys_c[pl.ds(j * L, L)] = jnp.where(
                is_min, jnp.minimum(a, b), jnp.maximum(a, b))
            return 0

        lax.fori_loop(0, NVC, body, 0)
        pltpu.sync_copy(keys_c, dst.at[pl.ds(base, CH)])
        plsc.subcore_barrier()

    def cross_aligned(src, dst, sc):
        part = lax.bitwise_xor(w, sc)
        is_min = jnp.broadcast_to(lax.bitwise_and(w, sc) == 0, (L,))
        pltpu.sync_copy(src.at[pl.ds(base, CH)], keys_c)
        pltpu.sync_copy(src.at[pl.ds(part * CH, CH)], pkeys)

        def body(j, _):
            a = keys_c[pl.ds(j * L, L)]
            b = pkeys[pl.ds(j * L, L)]
            keys_c[pl.ds(j * L, L)] = jnp.where(
                is_min, jnp.minimum(a, b), jnp.maximum(a, b))
            return 0

        lax.fori_loop(0, NVC, body, 0)
        pltpu.sync_copy(keys_c, dst.at[pl.ds(base, CH)])
        plsc.subcore_barrier()

    def local_finish(src, dst):
        pltpu.sync_copy(src.at[pl.ds(base, CH)], keys_c)
        _ce_stages(keys_c, CH, CH // 2)
        _sort_pass(keys_c, NVC)
        pltpu.sync_copy(keys_c, dst.at[pl.ds(base, CH)])
        plsc.subcore_barrier()

    pltpu.sync_copy(logits_hbm.at[pl.ds(base, CH)], lg)
    zero = jnp.zeros((L,), jnp.float32)
    for j in range(8):
        stats_my[j] = zero

    def sig_body(i, _):
        x = lg[pl.ds(i * L, L)]
        sv = _sigmoid(x)
        stats_my[2] = stats_my[2] + sv
        keys_c[pl.ds(i * L, L)] = sv
        return 0

    lax.fori_loop(0, NVC, sig_body, 0)

    _local_sort(keys_c, scratch, CH)
    pltpu.sync_copy(keys_c, sh_a.at[pl.ds(base, CH)])
    plsc.subcore_barrier()

    cross_mirror(sh_a, sh_b, 1)
    local_finish(sh_b, sh_a)
    cross_mirror(sh_a, sh_b, 2)
    cross_aligned(sh_b, sh_a, 1)
    local_finish(sh_a, sh_b)
    cross_mirror(sh_b, sh_a, 4)
    cross_aligned(sh_a, sh_b, 2)
    cross_aligned(sh_b, sh_a, 1)
    local_finish(sh_a, sh_b)

    pltpu.sync_copy(sh_b.at[pl.ds(N_ELEM, N_ELEM)], sfall.at[pl.ds(0, N_ELEM)])
    pltpu.sync_copy(sh_b.at[pl.ds(w * QB, QB)], lg.at[pl.ds(0, QB)])
    iota = lax.iota(jnp.int32, L)

    def search_body(v, _):
        q = lg[pl.ds(v * L, L)]
        pos = iota + (w * QB + v * L)
        cnt_le = _count_le(sfall, q)
        c_gt = (N_ELEM - cnt_le).astype(jnp.float32)
        r = (N_ELEM - pos).astype(jnp.float32)
        big_r = r + c_gt
        stats_my[3] = stats_my[3] + r / (big_r + 1e-08)
        stats_my[4] = stats_my[4] + big_r
        return 0

    lax.fori_loop(0, QB // L, search_body, 0)

    pltpu.sync_copy(stats_my, shared_stats.at[w])
    plsc.subcore_barrier()

    @pl.when(w == 0)
    def _():
        q05 = jnp.full((L,), 0.5, jnp.float32)
        le_f = _count_le(sfall, q05)
        lt_f = _count_le(sfall, q05, strict=True)
        pltpu.sync_copy(sh_b.at[pl.ds(0, N_ELEM)], sfall.at[pl.ds(0, N_ELEM)])
        le_r = _count_le(sfall, q05)
        lt_r = _count_le(sfall, q05, strict=True)
        n_pr = (N_ELEM - le_r).astype(jnp.float32)
        n_pf = (N_ELEM - le_f).astype(jnp.float32)
        eq_r = (le_r - lt_r).astype(jnp.float32)
        eq_f = (le_f - lt_f).astype(jnp.float32)

        pltpu.sync_copy(shared_stats, stats_all)

        def red(row, tiles):
            acc = jnp.zeros((L,), jnp.float32)
            for t in tiles:
                acc = acc + stats_all[t, row]
            return jnp.broadcast_to(jnp.sum(acc), (L,))

        reals = range(8)
        fakes = range(8, NT)
        sum_r = red(2, reals)
        sum_f = red(2, fakes)
        pr_sum = red(3, range(NT))
        r_sum = red(4, range(NT))

        p = jnp.full((L,), float(N_ELEM), jnp.float32)
        n = jnp.full((L,), float(N_ELEM), jnp.float32)
        accuracy = (n_pr + (n - n_pf)) / (p + n)
        tp = n_pr
        fp = n_pf
        fn = p - n_pr
        precision = tp / (tp + fp + 1e-08)
        recall = tp / (tp + fn + 1e-08)
        f1 = 2.0 * precision * recall / (precision + recall + 1e-08)
        pr_auc = pr_sum / (p + 1e-08)
        sum_pos_ranks = float((2 * N_ELEM + 1) * N_ELEM) - r_sum
        roc_auc = (sum_pos_ranks - float(N_ELEM * (N_ELEM + 1) // 2)) \
            / float(N_ELEM * N_ELEM)
        gap = sum_r / p - sum_f / n
        prob_fake = sum_f / n
        ppf = (n_pr + n_pf) / (p + n)
        acc05 = (eq_r / p + eq_f / n) * 0.5

        vals = [accuracy, precision, recall, f1, pr_auc, roc_auc,
                gap, prob_fake, ppf, acc05]
        ov = jnp.zeros((L,), jnp.float32)
        for i, s in enumerate(vals):
            ov = jnp.where(iota == i, s, ov)
        outv[...] = ov
        pltpu.sync_copy(outv, out_hbm)


@jax.jit
def _run(logits):
    mesh = plsc.VectorSubcoreMesh(
        core_axis_name="c", subcore_axis_name="s", num_cores=1)
    f = pl.kernel(
        _tec_body,
        out_type=jax.ShapeDtypeStruct((L,), jnp.float32),
        mesh=mesh,
        compiler_params=pltpu.CompilerParams(needs_layout_passes=False),
        scratch_types=[
            pltpu.VMEM((CH,), jnp.float32),
            pltpu.VMEM((CH,), jnp.float32),
            pltpu.VMEM((CH,), jnp.float32),
            pltpu.VMEM((CH // 2,), jnp.float32),
            pltpu.VMEM((N_ELEM + 256,), jnp.float32),
            pltpu.VMEM((8, L), jnp.float32),
            pltpu.VMEM((NT, 8, L), jnp.float32),
            pltpu.VMEM((L,), jnp.float32),
            pltpu.VMEM_SHARED((2 * N_ELEM,), jnp.float32),
            pltpu.VMEM_SHARED((2 * N_ELEM,), jnp.float32),
            pltpu.VMEM_SHARED((NT, 8, L), jnp.float32),
        ],
    )
    return f(logits)


def kernel(gt_logits, gen_logits):
    logits = jnp.concatenate(
        [jnp.squeeze(gt_logits, -1), jnp.squeeze(gen_logits, -1)], axis=0)
    out = _run(logits)
    (accuracy, precision, recall, f1, pr_auc, roc_auc,
     gap, prob_fake, ppf, acc05) = (out[i] for i in range(10))
    return (accuracy, precision, recall, f1, pr_auc, roc_auc,
            gap, prob_fake, ppf, acc05)


# speedup vs baseline: 1.8918x; 1.6484x over previous
import jax
import jax.numpy as jnp
from jax import lax
from jax.experimental import pallas as pl
from jax.experimental.pallas import tpu as pltpu
from jax.experimental.pallas import tpu_sc as plsc

N_ELEM = 16384
NT = 16
CH = 2048
NVC = CH // 16
QB = N_ELEM // NT
L = 16


def _log2(x):
    return x.bit_length() - 1


def _sigmoid(x):
    e = jnp.exp(-jnp.abs(x))
    return jnp.where(x >= 0, 1.0 / (1.0 + e), e / (1.0 + e))


UNROLL = 8


def _sort_pass(buf, n_vregs):
    u = UNROLL if n_vregs % UNROLL == 0 else 1

    def body(i, _):
        for j in range(u):
            x = buf[pl.ds((i * u + j) * L, L)]
            sk, _unused = plsc.sort_key_val(x, x)
            buf[pl.ds((i * u + j) * L, L)] = sk
        return 0

    lax.fori_loop(0, n_vregs // u, body, 0)


def _ce_stages(buf, size, s_hi):
    s = s_hi
    n_units = size // (2 * L)
    while s >= 16:
        sv = s // L
        shift = _log2(sv)
        mask = sv - 1
        uu = UNROLL if n_units % UNROLL == 0 else 1

        def body(i, _, s=s, shift=shift, mask=mask, sv=sv, uu=uu):
            for j in range(uu):
                u = i * uu + j
                g = lax.shift_right_logical(u, shift)
                r = lax.bitwise_and(u, mask)
                b1 = (g * (2 * sv) + r) * L
                b2 = b1 + s
                a = buf[pl.ds(b1, L)]
                b = buf[pl.ds(b2, L)]
                buf[pl.ds(b1, L)] = jnp.minimum(a, b)
                buf[pl.ds(b2, L)] = jnp.maximum(a, b)
            return 0

        lax.fori_loop(0, n_units // uu, body, 0)
        s //= 2


def _local_sort(buf, scratch, size):
    nv_all = size // L
    _sort_pass(buf, nv_all)
    m = 16
    while m < size:
        nv = m // L
        npairs = size // (2 * m)

        def rev_body(p, _, nv=nv, m=m):
            base = p * 2 * m + m
            for j in range(nv):
                scratch[pl.ds(j * L, L)] = jnp.flip(
                    buf[pl.ds(base + (nv - 1 - j) * L, L)], 0)
            for j in range(nv):
                buf[pl.ds(base + j * L, L)] = scratch[pl.ds(j * L, L)]
            return 0

        lax.fori_loop(0, npairs, rev_body, 0)
        _ce_stages(buf, size, m)
        _sort_pass(buf, nv_all)
        m *= 2


def _count_le(arr, q, strict=False):
    lo = jnp.zeros((L,), jnp.int32)
    s = N_ELEM // 2
    while s >= 1:
        idx = lo + (s - 1)
        v = plsc.load_gather(arr, [idx])
        take = (v < q) if strict else (v <= q)
        lo = jnp.where(take, lo + s, lo)
        s //= 2
    return lo


def _tec_body(logits_hbm, out_hbm,
              lg, keys_c, pkeys, scratch, sfall, stats_my, stats_all, outv,
              sh_a, sh_b, shared_stats):
    w = lax.axis_index("s")
    base = w * CH

    def cross_mirror(src, dst, k):
        c = lax.bitwise_and(w, 7)
        g = lax.bitwise_and(c, 2 * k - 1)
        part_l = (c - g) + (2 * k - 1 - g)
        part = lax.bitwise_or(lax.bitwise_and(w, 8), part_l)
        is_min = jnp.broadcast_to(g < k, (L,))
        pltpu.sync_copy(src.at[pl.ds(base, CH)], keys_c)
        pltpu.sync_copy(src.at[pl.ds(part * CH, CH)], pkeys)

        def body(i, _):
            for jj in range(UNROLL):
                j = i * UNROLL + jj
                a = keys_c[pl.ds(j * L, L)]
                b = jnp.flip(pkeys[pl.ds((NVC - 1) * L - j * L, L)], 0)
                keys_c[pl.ds(j * L, L)] = jnp.where(
                    is_min, jnp.minimum(a, b), jnp.maximum(a, b))
            return 0

        lax.fori_loop(0, NVC // UNROLL, body, 0)
        pltpu.sync_copy(keys_c, dst.at[pl.ds(base, CH)])
        plsc.subcore_barrier()

    def cross_aligned(src, dst, sc):
        part = lax.bitwise_xor(w, sc)
        is_min = jnp.broadcast_to(lax.bitwise_and(w, sc) == 0, (L,))
        pltpu.sync_copy(src.at[pl.ds(base, CH)], keys_c)
        pltpu.sync_copy(src.at[pl.ds(part * CH, CH)], pkeys)

        def body(i, _):
            for jj in range(UNROLL):
                j = i * UNROLL + jj
                a = keys_c[pl.ds(j * L, L)]
                b = pkeys[pl.ds(j * L, L)]
                keys_c[pl.ds(j * L, L)] = jnp.where(
                    is_min, jnp.minimum(a, b), jnp.maximum(a, b))
            return 0

        lax.fori_loop(0, NVC // UNROLL, body, 0)
        pltpu.sync_copy(keys_c, dst.at[pl.ds(base, CH)])
        plsc.subcore_barrier()

    def local_finish(src, dst):
        pltpu.sync_copy(src.at[pl.ds(base, CH)], keys_c)
        _ce_stages(keys_c, CH, CH // 2)
        _sort_pass(keys_c, NVC)
        pltpu.sync_copy(keys_c, dst.at[pl.ds(base, CH)])
        plsc.subcore_barrier()

    pltpu.sync_copy(logits_hbm.at[pl.ds(base, CH)], lg)
    zero = jnp.zeros((L,), jnp.float32)
    for j in range(8):
        stats_my[j] = zero

    def sig_body(i, _):
        acc = jnp.zeros((L,), jnp.float32)
        for jj in range(UNROLL):
            j = i * UNROLL + jj
            x = lg[pl.ds(j * L, L)]
            sv = _sigmoid(x)
            acc = acc + sv
            keys_c[pl.ds(j * L, L)] = sv
        stats_my[2] = stats_my[2] + acc
        return 0

    lax.fori_loop(0, NVC // UNROLL, sig_body, 0)

    _local_sort(keys_c, scratch, CH)
    pltpu.sync_copy(keys_c, sh_a.at[pl.ds(base, CH)])
    plsc.subcore_barrier()

    cross_mirror(sh_a, sh_b, 1)
    local_finish(sh_b, sh_a)
    cross_mirror(sh_a, sh_b, 2)
    cross_aligned(sh_b, sh_a, 1)
    local_finish(sh_a, sh_b)
    cross_mirror(sh_b, sh_a, 4)
    cross_aligned(sh_a, sh_b, 2)
    cross_aligned(sh_b, sh_a, 1)
    local_finish(sh_a, sh_b)

    pltpu.sync_copy(sh_b.at[pl.ds(N_ELEM, N_ELEM)], sfall.at[pl.ds(0, N_ELEM)])
    pltpu.sync_copy(sh_b.at[pl.ds(w * QB, QB)], lg.at[pl.ds(0, QB)])
    iota = lax.iota(jnp.int32, L)

    def search_body(i, _):
        pr_t = jnp.zeros((L,), jnp.float32)
        rr_t = jnp.zeros((L,), jnp.float32)
        for jj in range(4):
            v = i * 4 + jj
            q = lg[pl.ds(v * L, L)]
            pos = iota + (w * QB + v * L)
            cnt_le = _count_le(sfall, q)
            c_gt = (N_ELEM - cnt_le).astype(jnp.float32)
            r = (N_ELEM - pos).astype(jnp.float32)
            big_r = r + c_gt
            pr_t = pr_t + r / (big_r + 1e-08)
            rr_t = rr_t + big_r
        stats_my[3] = stats_my[3] + pr_t
        stats_my[4] = stats_my[4] + rr_t
        return 0

    lax.fori_loop(0, QB // L // 4, search_body, 0)

    pltpu.sync_copy(stats_my, shared_stats.at[w])
    plsc.subcore_barrier()

    @pl.when(w == 0)
    def _():
        q05 = jnp.full((L,), 0.5, jnp.float32)
        le_f = _count_le(sfall, q05)
        lt_f = _count_le(sfall, q05, strict=True)
        pltpu.sync_copy(sh_b.at[pl.ds(0, N_ELEM)], sfall.at[pl.ds(0, N_ELEM)])
        le_r = _count_le(sfall, q05)
        lt_r = _count_le(sfall, q05, strict=True)
        n_pr = (N_ELEM - le_r).astype(jnp.float32)
        n_pf = (N_ELEM - le_f).astype(jnp.float32)
        eq_r = (le_r - lt_r).astype(jnp.float32)
        eq_f = (le_f - lt_f).astype(jnp.float32)

        pltpu.sync_copy(shared_stats, stats_all)

        def red(row, tiles):
            acc = jnp.zeros((L,), jnp.float32)
            for t in tiles:
                acc = acc + stats_all[t, row]
            return jnp.broadcast_to(jnp.sum(acc), (L,))

        reals = range(8)
        fakes = range(8, NT)
        sum_r = red(2, reals)
        sum_f = red(2, fakes)
        pr_sum = red(3, range(NT))
        r_sum = red(4, range(NT))

        p = jnp.full((L,), float(N_ELEM), jnp.float32)
        n = jnp.full((L,), float(N_ELEM), jnp.float32)
        accuracy = (n_pr + (n - n_pf)) / (p + n)
        tp = n_pr
        fp = n_pf
        fn = p - n_pr
        precision = tp / (tp + fp + 1e-08)
        recall = tp / (tp + fn + 1e-08)
        f1 = 2.0 * precision * recall / (precision + recall + 1e-08)
        pr_auc = pr_sum / (p + 1e-08)
        sum_pos_ranks = float((2 * N_ELEM + 1) * N_ELEM) - r_sum
        roc_auc = (sum_pos_ranks - float(N_ELEM * (N_ELEM + 1) // 2)) \
            / float(N_ELEM * N_ELEM)
        gap = sum_r / p - sum_f / n
        prob_fake = sum_f / n
        ppf = (n_pr + n_pf) / (p + n)
        acc05 = (eq_r / p + eq_f / n) * 0.5

        vals = [accuracy, precision, recall, f1, pr_auc, roc_auc,
                gap, prob_fake, ppf, acc05]
        ov = jnp.zeros((L,), jnp.float32)
        for i, s in enumerate(vals):
            ov = jnp.where(iota == i, s, ov)
        outv[...] = ov
        pltpu.sync_copy(outv, out_hbm)


@jax.jit
def _run(logits):
    mesh = plsc.VectorSubcoreMesh(
        core_axis_name="c", subcore_axis_name="s", num_cores=1)
    f = pl.kernel(
        _tec_body,
        out_type=jax.ShapeDtypeStruct((L,), jnp.float32),
        mesh=mesh,
        compiler_params=pltpu.CompilerParams(needs_layout_passes=False),
        scratch_types=[
            pltpu.VMEM((CH,), jnp.float32),
            pltpu.VMEM((CH,), jnp.float32),
            pltpu.VMEM((CH,), jnp.float32),
            pltpu.VMEM((CH // 2,), jnp.float32),
            pltpu.VMEM((N_ELEM + 256,), jnp.float32),
            pltpu.VMEM((8, L), jnp.float32),
            pltpu.VMEM((NT, 8, L), jnp.float32),
            pltpu.VMEM((L,), jnp.float32),
            pltpu.VMEM_SHARED((2 * N_ELEM,), jnp.float32),
            pltpu.VMEM_SHARED((2 * N_ELEM,), jnp.float32),
            pltpu.VMEM_SHARED((NT, 8, L), jnp.float32),
        ],
    )
    return f(logits)


def kernel(gt_logits, gen_logits):
    logits = jnp.concatenate(
        [jnp.squeeze(gt_logits, -1), jnp.squeeze(gen_logits, -1)], axis=0)
    out = _run(logits)
    (accuracy, precision, recall, f1, pr_auc, roc_auc,
     gap, prob_fake, ppf, acc05) = (out[i] for i in range(10))
    return (accuracy, precision, recall, f1, pr_auc, roc_auc,
            gap, prob_fake, ppf, acc05)


# speedup vs baseline: 1.9205x; 1.0152x over previous
import jax
import jax.numpy as jnp
from jax import lax
from jax.experimental import pallas as pl
from jax.experimental.pallas import tpu as pltpu
from jax.experimental.pallas import tpu_sc as plsc

N_ELEM = 16384
NT = 16
CH = 2048
NVC = CH // 16
QB = N_ELEM // NT
L = 16


def _log2(x):
    return x.bit_length() - 1


def _sigmoid(x):
    e = jnp.exp(-jnp.abs(x))
    return jnp.where(x >= 0, 1.0 / (1.0 + e), e / (1.0 + e))


UNROLL = 8


def _sort_pass(buf, n_vregs):
    u = UNROLL if n_vregs % UNROLL == 0 else 1

    def body(i, _):
        for j in range(u):
            x = buf[pl.ds((i * u + j) * L, L)]
            sk, _unused = plsc.sort_key_val(x, x)
            buf[pl.ds((i * u + j) * L, L)] = sk
        return 0

    lax.fori_loop(0, n_vregs // u, body, 0)


def _ce_stages(buf, size, s_hi):
    s = s_hi
    n_units = size // (2 * L)
    while s >= 16:
        sv = s // L
        shift = _log2(sv)
        mask = sv - 1
        uu = UNROLL if n_units % UNROLL == 0 else 1

        def body(i, _, s=s, shift=shift, mask=mask, sv=sv, uu=uu):
            for j in range(uu):
                u = i * uu + j
                g = lax.shift_right_logical(u, shift)
                r = lax.bitwise_and(u, mask)
                b1 = (g * (2 * sv) + r) * L
                b2 = b1 + s
                a = buf[pl.ds(b1, L)]
                b = buf[pl.ds(b2, L)]
                buf[pl.ds(b1, L)] = jnp.minimum(a, b)
                buf[pl.ds(b2, L)] = jnp.maximum(a, b)
            return 0

        lax.fori_loop(0, n_units // uu, body, 0)
        s //= 2


def _local_sort(buf, scratch, size):
    nv_all = size // L
    _sort_pass(buf, nv_all)
    m = 16
    while m < size:
        nv = m // L
        npairs = size // (2 * m)

        def rev_body(p, _, nv=nv, m=m):
            base = p * 2 * m + m
            for j in range(nv):
                scratch[pl.ds(j * L, L)] = jnp.flip(
                    buf[pl.ds(base + (nv - 1 - j) * L, L)], 0)
            for j in range(nv):
                buf[pl.ds(base + j * L, L)] = scratch[pl.ds(j * L, L)]
            return 0

        lax.fori_loop(0, npairs, rev_body, 0)
        _ce_stages(buf, size, m)
        _sort_pass(buf, nv_all)
        m *= 2


def _count_le(arr, q, strict=False):
    lo = jnp.zeros((L,), jnp.int32)
    s = N_ELEM // 2
    while s >= 1:
        idx = lo + (s - 1)
        v = plsc.load_gather(arr, [idx])
        take = (v < q) if strict else (v <= q)
        lo = jnp.where(take, lo + s, lo)
        s //= 2
    return lo


def _tec_body(logits_hbm, out_hbm,
              lg, keys_c, pkeys, scratch, sfall, stats_my, stats_all, outv,
              sh_a, sh_b, shared_stats):
    w = lax.axis_index("s")
    base = w * CH

    def cross_mirror(src, dst, k, finish=False):
        c = lax.bitwise_and(w, 7)
        g = lax.bitwise_and(c, 2 * k - 1)
        part_l = (c - g) + (2 * k - 1 - g)
        part = lax.bitwise_or(lax.bitwise_and(w, 8), part_l)
        is_min = jnp.broadcast_to(g < k, (L,))
        pltpu.sync_copy(src.at[pl.ds(base, CH)], keys_c)
        pltpu.sync_copy(src.at[pl.ds(part * CH, CH)], pkeys)

        def body(i, _):
            for jj in range(UNROLL):
                j = i * UNROLL + jj
                a = keys_c[pl.ds(j * L, L)]
                b = jnp.flip(pkeys[pl.ds((NVC - 1) * L - j * L, L)], 0)
                keys_c[pl.ds(j * L, L)] = jnp.where(
                    is_min, jnp.minimum(a, b), jnp.maximum(a, b))
            return 0

        lax.fori_loop(0, NVC // UNROLL, body, 0)
        if finish:
            _ce_stages(keys_c, CH, CH // 2)
            _sort_pass(keys_c, NVC)
        pltpu.sync_copy(keys_c, dst.at[pl.ds(base, CH)])
        plsc.subcore_barrier()

    def cross_aligned(src, dst, sc, finish=False):
        part = lax.bitwise_xor(w, sc)
        is_min = jnp.broadcast_to(lax.bitwise_and(w, sc) == 0, (L,))
        pltpu.sync_copy(src.at[pl.ds(base, CH)], keys_c)
        pltpu.sync_copy(src.at[pl.ds(part * CH, CH)], pkeys)

        def body(i, _):
            for jj in range(UNROLL):
                j = i * UNROLL + jj
                a = keys_c[pl.ds(j * L, L)]
                b = pkeys[pl.ds(j * L, L)]
                keys_c[pl.ds(j * L, L)] = jnp.where(
                    is_min, jnp.minimum(a, b), jnp.maximum(a, b))
            return 0

        lax.fori_loop(0, NVC // UNROLL, body, 0)
        if finish:
            _ce_stages(keys_c, CH, CH // 2)
            _sort_pass(keys_c, NVC)
        pltpu.sync_copy(keys_c, dst.at[pl.ds(base, CH)])
        plsc.subcore_barrier()

    pltpu.sync_copy(logits_hbm.at[pl.ds(base, CH)], lg)
    zero = jnp.zeros((L,), jnp.float32)
    for j in range(8):
        stats_my[j] = zero

    def sig_body(i, _):
        acc = jnp.zeros((L,), jnp.float32)
        for jj in range(UNROLL):
            j = i * UNROLL + jj
            x = lg[pl.ds(j * L, L)]
            sv = _sigmoid(x)
            acc = acc + sv
            keys_c[pl.ds(j * L, L)] = sv
        stats_my[2] = stats_my[2] + acc
        return 0

    lax.fori_loop(0, NVC // UNROLL, sig_body, 0)

    _local_sort(keys_c, scratch, CH)
    pltpu.sync_copy(keys_c, sh_a.at[pl.ds(base, CH)])
    plsc.subcore_barrier()

    cross_mirror(sh_a, sh_b, 1, finish=True)
    cross_mirror(sh_b, sh_a, 2)
    cross_aligned(sh_a, sh_b, 1, finish=True)
    cross_mirror(sh_b, sh_a, 4)
    cross_aligned(sh_a, sh_b, 2)
    cross_aligned(sh_b, sh_a, 1, finish=True)

    pltpu.sync_copy(sh_a.at[pl.ds(N_ELEM, N_ELEM)], sfall.at[pl.ds(0, N_ELEM)])
    pltpu.sync_copy(sh_a.at[pl.ds(w * QB, QB)], lg.at[pl.ds(0, QB)])
    iota = lax.iota(jnp.int32, L)

    def search_body(i, _):
        pr_t = jnp.zeros((L,), jnp.float32)
        rr_t = jnp.zeros((L,), jnp.float32)
        for jj in range(4):
            v = i * 4 + jj
            q = lg[pl.ds(v * L, L)]
            pos = iota + (w * QB + v * L)
            cnt_le = _count_le(sfall, q)
            c_gt = (N_ELEM - cnt_le).astype(jnp.float32)
            r = (N_ELEM - pos).astype(jnp.float32)
            big_r = r + c_gt
            pr_t = pr_t + r / (big_r + 1e-08)
            rr_t = rr_t + big_r
        stats_my[3] = stats_my[3] + pr_t
        stats_my[4] = stats_my[4] + rr_t
        return 0

    lax.fori_loop(0, QB // L // 4, search_body, 0)

    pltpu.sync_copy(stats_my, shared_stats.at[w])
    plsc.subcore_barrier()

    @pl.when(w == 0)
    def _():
        q05 = jnp.full((L,), 0.5, jnp.float32)
        le_f = _count_le(sfall, q05)
        lt_f = _count_le(sfall, q05, strict=True)
        pltpu.sync_copy(sh_a.at[pl.ds(0, N_ELEM)], sfall.at[pl.ds(0, N_ELEM)])
        le_r = _count_le(sfall, q05)
        lt_r = _count_le(sfall, q05, strict=True)
        n_pr = (N_ELEM - le_r).astype(jnp.float32)
        n_pf = (N_ELEM - le_f).astype(jnp.float32)
        eq_r = (le_r - lt_r).astype(jnp.float32)
        eq_f = (le_f - lt_f).astype(jnp.float32)

        pltpu.sync_copy(shared_stats, stats_all)

        def red(row, tiles):
            acc = jnp.zeros((L,), jnp.float32)
            for t in tiles:
                acc = acc + stats_all[t, row]
            return jnp.broadcast_to(jnp.sum(acc), (L,))

        reals = range(8)
        fakes = range(8, NT)
        sum_r = red(2, reals)
        sum_f = red(2, fakes)
        pr_sum = red(3, range(NT))
        r_sum = red(4, range(NT))

        p = jnp.full((L,), float(N_ELEM), jnp.float32)
        n = jnp.full((L,), float(N_ELEM), jnp.float32)
        accuracy = (n_pr + (n - n_pf)) / (p + n)
        tp = n_pr
        fp = n_pf
        fn = p - n_pr
        precision = tp / (tp + fp + 1e-08)
        recall = tp / (tp + fn + 1e-08)
        f1 = 2.0 * precision * recall / (precision + recall + 1e-08)
        pr_auc = pr_sum / (p + 1e-08)
        sum_pos_ranks = float((2 * N_ELEM + 1) * N_ELEM) - r_sum
        roc_auc = (sum_pos_ranks - float(N_ELEM * (N_ELEM + 1) // 2)) \
            / float(N_ELEM * N_ELEM)
        gap = sum_r / p - sum_f / n
        prob_fake = sum_f / n
        ppf = (n_pr + n_pf) / (p + n)
        acc05 = (eq_r / p + eq_f / n) * 0.5

        vals = [accuracy, precision, recall, f1, pr_auc, roc_auc,
                gap, prob_fake, ppf, acc05]
        ov = jnp.zeros((L,), jnp.float32)
        for i, s in enumerate(vals):
            ov = jnp.where(iota == i, s, ov)
        outv[...] = ov
        pltpu.sync_copy(outv, out_hbm)


@jax.jit
def _run(logits):
    mesh = plsc.VectorSubcoreMesh(
        core_axis_name="c", subcore_axis_name="s", num_cores=1)
    f = pl.kernel(
        _tec_body,
        out_type=jax.ShapeDtypeStruct((L,), jnp.float32),
        mesh=mesh,
        compiler_params=pltpu.CompilerParams(needs_layout_passes=False),
        scratch_types=[
            pltpu.VMEM((CH,), jnp.float32),
            pltpu.VMEM((CH,), jnp.float32),
            pltpu.VMEM((CH,), jnp.float32),
            pltpu.VMEM((CH // 2,), jnp.float32),
            pltpu.VMEM((N_ELEM + 256,), jnp.float32),
            pltpu.VMEM((8, L), jnp.float32),
            pltpu.VMEM((NT, 8, L), jnp.float32),
            pltpu.VMEM((L,), jnp.float32),
            pltpu.VMEM_SHARED((2 * N_ELEM,), jnp.float32),
            pltpu.VMEM_SHARED((2 * N_ELEM,), jnp.float32),
            pltpu.VMEM_SHARED((NT, 8, L), jnp.float32),
        ],
    )
    return f(logits)


def kernel(gt_logits, gen_logits):
    logits = jnp.concatenate(
        [jnp.squeeze(gt_logits, -1), jnp.squeeze(gen_logits, -1)], axis=0)
    out = _run(logits)
    (accuracy, precision, recall, f1, pr_auc, roc_auc,
     gap, prob_fake, ppf, acc05) = (out[i] for i in range(10))
    return (accuracy, precision, recall, f1, pr_auc, roc_auc,
            gap, prob_fake, ppf, acc05)


# speedup vs baseline: 2.1590x; 1.1242x over previous
import jax
import jax.numpy as jnp
from jax import lax
from jax.experimental import pallas as pl
from jax.experimental.pallas import tpu as pltpu
from jax.experimental.pallas import tpu_sc as plsc

N_ELEM = 16384
NT = 16
CH = 2048
NVC = CH // 16
QB = N_ELEM // NT
L = 16


def _log2(x):
    return x.bit_length() - 1


def _sigmoid(x):
    e = jnp.exp(-jnp.abs(x))
    return jnp.where(x >= 0, 1.0 / (1.0 + e), e / (1.0 + e))


UNROLL = 8


def _sort_pass(buf, n_vregs):
    u = UNROLL if n_vregs % UNROLL == 0 else 1

    def body(i, _):
        for j in range(u):
            x = buf[pl.ds((i * u + j) * L, L)]
            sk, _unused = plsc.sort_key_val(x, x)
            buf[pl.ds((i * u + j) * L, L)] = sk
        return 0

    lax.fori_loop(0, n_vregs // u, body, 0)


def _ce_stages(buf, size, s_hi, s_lo=16):
    s = s_hi
    n_units = size // (2 * L)
    while s >= s_lo:
        sv = s // L
        shift = _log2(sv)
        mask = sv - 1
        uu = UNROLL if n_units % UNROLL == 0 else 1

        def body(i, _, s=s, shift=shift, mask=mask, sv=sv, uu=uu):
            for j in range(uu):
                u = i * uu + j
                g = lax.shift_right_logical(u, shift)
                r = lax.bitwise_and(u, mask)
                b1 = (g * (2 * sv) + r) * L
                b2 = b1 + s
                a = buf[pl.ds(b1, L)]
                b = buf[pl.ds(b2, L)]
                buf[pl.ds(b1, L)] = jnp.minimum(a, b)
                buf[pl.ds(b2, L)] = jnp.maximum(a, b)
            return 0

        lax.fori_loop(0, n_units // uu, body, 0)
        s //= 2


def _fused_level(buf, size, m):
    nb = 2 * m // L
    nhalf = nb // 2
    nblocks = size // (2 * m)
    bpi = max(1, 8 // nb)

    def body(i, _, nb=nb, nhalf=nhalf, bpi=bpi):
        for bb in range(bpi):
            base_v = (i * bpi + bb) * nb
            v = [buf[pl.ds((base_v + j) * L, L)] for j in range(nb)]
            for j in range(nhalf):
                bflip = jnp.flip(v[nb - 1 - j], 0)
                lo = jnp.minimum(v[j], bflip)
                hi = jnp.maximum(v[j], bflip)
                v[j] = lo
                v[nb - 1 - j] = jnp.flip(hi, 0)
            sv = nb // 4
            while sv >= 1:
                for g0 in range(0, nb, 2 * sv):
                    for j in range(sv):
                        a = v[g0 + j]
                        b = v[g0 + j + sv]
                        v[g0 + j] = jnp.minimum(a, b)
                        v[g0 + j + sv] = jnp.maximum(a, b)
                sv //= 2
            for j in range(nb):
                sk, _unused = plsc.sort_key_val(v[j], v[j])
                buf[pl.ds((base_v + j) * L, L)] = sk
        return 0

    lax.fori_loop(0, nblocks // bpi, body, 0)


def _mirror_pass(buf, size, m):
    nv = m // L
    shift = _log2(nv)
    mask = nv - 1
    n_units = size // (2 * L)
    uu = UNROLL if n_units % UNROLL == 0 else 1

    def body(i, _):
        for jj in range(uu):
            u = i * uu + jj
            p = lax.shift_right_logical(u, shift)
            j = lax.bitwise_and(u, mask)
            ai = (p * 2 * nv + j) * L
            bi = (p * 2 * nv + 2 * nv - 1 - j) * L
            a = buf[pl.ds(ai, L)]
            b = jnp.flip(buf[pl.ds(bi, L)], 0)
            buf[pl.ds(ai, L)] = jnp.minimum(a, b)
            buf[pl.ds(bi, L)] = jnp.flip(jnp.maximum(a, b), 0)
        return 0

    lax.fori_loop(0, n_units // uu, body, 0)


def _tail8(buf, size):
    def body(i, _):
        base_v = i * 8
        v = [buf[pl.ds((base_v + j) * L, L)] for j in range(8)]
        for sv in (4, 2, 1):
            for g0 in range(0, 8, 2 * sv):
                for j in range(sv):
                    a = v[g0 + j]
                    b = v[g0 + j + sv]
                    v[g0 + j] = jnp.minimum(a, b)
                    v[g0 + j + sv] = jnp.maximum(a, b)
        for j in range(8):
            sk, _unused = plsc.sort_key_val(v[j], v[j])
            buf[pl.ds((base_v + j) * L, L)] = sk
        return 0

    lax.fori_loop(0, size // (8 * L), body, 0)


def _local_sort(buf, size):
    _sort_pass(buf, size // L)
    m = 16
    while m < size:
        if m <= 128:
            _fused_level(buf, size, m)
        else:
            _mirror_pass(buf, size, m)
            _ce_stages(buf, size, m // 2, s_lo=128)
            _tail8(buf, size)
        m *= 2


def _count_le(arr, q, strict=False):
    lo = jnp.zeros((L,), jnp.int32)
    s = N_ELEM // 2
    while s >= 1:
        idx = lo + (s - 1)
        v = plsc.load_gather(arr, [idx])
        take = (v < q) if strict else (v <= q)
        lo = jnp.where(take, lo + s, lo)
        s //= 2
    return lo


def _tec_body(logits_hbm, out_hbm,
              lg, keys_c, pkeys, sfall, stats_my, stats_all, outv,
              sh_a, sh_b, shared_stats):
    w = lax.axis_index("s")
    base = w * CH

    def cross_mirror(src, dst, k, finish=False):
        c = lax.bitwise_and(w, 7)
        g = lax.bitwise_and(c, 2 * k - 1)
        part_l = (c - g) + (2 * k - 1 - g)
        part = lax.bitwise_or(lax.bitwise_and(w, 8), part_l)
        is_min = jnp.broadcast_to(g < k, (L,))
        pltpu.sync_copy(src.at[pl.ds(base, CH)], keys_c)
        pltpu.sync_copy(src.at[pl.ds(part * CH, CH)], pkeys)

        def body(i, _):
            for jj in range(UNROLL):
                j = i * UNROLL + jj
                a = keys_c[pl.ds(j * L, L)]
                b = jnp.flip(pkeys[pl.ds((NVC - 1) * L - j * L, L)], 0)
                keys_c[pl.ds(j * L, L)] = jnp.where(
                    is_min, jnp.minimum(a, b), jnp.maximum(a, b))
            return 0

        lax.fori_loop(0, NVC // UNROLL, body, 0)
        if finish:
            _ce_stages(keys_c, CH, CH // 2, s_lo=128)
            _tail8(keys_c, CH)
        pltpu.sync_copy(keys_c, dst.at[pl.ds(base, CH)])
        plsc.subcore_barrier()

    def cross_aligned(src, dst, sc, finish=False):
        part = lax.bitwise_xor(w, sc)
        is_min = jnp.broadcast_to(lax.bitwise_and(w, sc) == 0, (L,))
        pltpu.sync_copy(src.at[pl.ds(base, CH)], keys_c)
        pltpu.sync_copy(src.at[pl.ds(part * CH, CH)], pkeys)

        def body(i, _):
            for jj in range(UNROLL):
                j = i * UNROLL + jj
                a = keys_c[pl.ds(j * L, L)]
                b = pkeys[pl.ds(j * L, L)]
                keys_c[pl.ds(j * L, L)] = jnp.where(
                    is_min, jnp.minimum(a, b), jnp.maximum(a, b))
            return 0

        lax.fori_loop(0, NVC // UNROLL, body, 0)
        if finish:
            _ce_stages(keys_c, CH, CH // 2, s_lo=128)
            _tail8(keys_c, CH)
        pltpu.sync_copy(keys_c, dst.at[pl.ds(base, CH)])
        plsc.subcore_barrier()

    pltpu.sync_copy(logits_hbm.at[pl.ds(base, CH)], lg)
    zero = jnp.zeros((L,), jnp.float32)
    for j in range(8):
        stats_my[j] = zero

    def sig_body(i, _):
        acc = jnp.zeros((L,), jnp.float32)
        for jj in range(UNROLL):
            j = i * UNROLL + jj
            x = lg[pl.ds(j * L, L)]
            sv = _sigmoid(x)
            acc = acc + sv
            keys_c[pl.ds(j * L, L)] = sv
        stats_my[2] = stats_my[2] + acc
        return 0

    lax.fori_loop(0, NVC // UNROLL, sig_body, 0)

    _local_sort(keys_c, CH)
    pltpu.sync_copy(keys_c, sh_a.at[pl.ds(base, CH)])
    plsc.subcore_barrier()

    cross_mirror(sh_a, sh_b, 1, finish=True)
    cross_mirror(sh_b, sh_a, 2)
    cross_aligned(sh_a, sh_b, 1, finish=True)
    cross_mirror(sh_b, sh_a, 4)
    cross_aligned(sh_a, sh_b, 2)
    cross_aligned(sh_b, sh_a, 1, finish=True)

    pltpu.sync_copy(sh_a.at[pl.ds(N_ELEM, N_ELEM)], sfall.at[pl.ds(0, N_ELEM)])
    pltpu.sync_copy(sh_a.at[pl.ds(w * QB, QB)], lg.at[pl.ds(0, QB)])
    iota = lax.iota(jnp.int32, L)

    def search_body(i, _):
        pr_t = jnp.zeros((L,), jnp.float32)
        rr_t = jnp.zeros((L,), jnp.float32)
        for jj in range(4):
            v = i * 4 + jj
            q = lg[pl.ds(v * L, L)]
            pos = iota + (w * QB + v * L)
            cnt_le = _count_le(sfall, q)
            c_gt = (N_ELEM - cnt_le).astype(jnp.float32)
            r = (N_ELEM - pos).astype(jnp.float32)
            big_r = r + c_gt
            pr_t = pr_t + r / (big_r + 1e-08)
            rr_t = rr_t + big_r
        stats_my[3] = stats_my[3] + pr_t
        stats_my[4] = stats_my[4] + rr_t
        return 0

    lax.fori_loop(0, QB // L // 4, search_body, 0)

    pltpu.sync_copy(stats_my, shared_stats.at[w])
    plsc.subcore_barrier()

    @pl.when(w == 0)
    def _():
        q05 = jnp.full((L,), 0.5, jnp.float32)
        le_f = _count_le(sfall, q05)
        lt_f = _count_le(sfall, q05, strict=True)
        pltpu.sync_copy(sh_a.at[pl.ds(0, N_ELEM)], sfall.at[pl.ds(0, N_ELEM)])
        le_r = _count_le(sfall, q05)
        lt_r = _count_le(sfall, q05, strict=True)
        n_pr = (N_ELEM - le_r).astype(jnp.float32)
        n_pf = (N_ELEM - le_f).astype(jnp.float32)
        eq_r = (le_r - lt_r).astype(jnp.float32)
        eq_f = (le_f - lt_f).astype(jnp.float32)

        pltpu.sync_copy(shared_stats, stats_all)

        def red(row, tiles):
            acc = jnp.zeros((L,), jnp.float32)
            for t in tiles:
                acc = acc + stats_all[t, row]
            return jnp.broadcast_to(jnp.sum(acc), (L,))

        reals = range(8)
        fakes = range(8, NT)
        sum_r = red(2, reals)
        sum_f = red(2, fakes)
        pr_sum = red(3, range(NT))
        r_sum = red(4, range(NT))

        p = jnp.full((L,), float(N_ELEM), jnp.float32)
        n = jnp.full((L,), float(N_ELEM), jnp.float32)
        accuracy = (n_pr + (n - n_pf)) / (p + n)
        tp = n_pr
        fp = n_pf
        fn = p - n_pr
        precision = tp / (tp + fp + 1e-08)
        recall = tp / (tp + fn + 1e-08)
        f1 = 2.0 * precision * recall / (precision + recall + 1e-08)
        pr_auc = pr_sum / (p + 1e-08)
        sum_pos_ranks = float((2 * N_ELEM + 1) * N_ELEM) - r_sum
        roc_auc = (sum_pos_ranks - float(N_ELEM * (N_ELEM + 1) // 2)) \
            / float(N_ELEM * N_ELEM)
        gap = sum_r / p - sum_f / n
        prob_fake = sum_f / n
        ppf = (n_pr + n_pf) / (p + n)
        acc05 = (eq_r / p + eq_f / n) * 0.5

        vals = [accuracy, precision, recall, f1, pr_auc, roc_auc,
                gap, prob_fake, ppf, acc05]
        ov = jnp.zeros((L,), jnp.float32)
        for i, s in enumerate(vals):
            ov = jnp.where(iota == i, s, ov)
        outv[...] = ov
        pltpu.sync_copy(outv, out_hbm)


@jax.jit
def _run(logits):
    mesh = plsc.VectorSubcoreMesh(
        core_axis_name="c", subcore_axis_name="s", num_cores=1)
    f = pl.kernel(
        _tec_body,
        out_type=jax.ShapeDtypeStruct((L,), jnp.float32),
        mesh=mesh,
        compiler_params=pltpu.CompilerParams(needs_layout_passes=False),
        scratch_types=[
            pltpu.VMEM((CH,), jnp.float32),
            pltpu.VMEM((CH,), jnp.float32),
            pltpu.VMEM((CH,), jnp.float32),
            pltpu.VMEM((N_ELEM + 256,), jnp.float32),
            pltpu.VMEM((8, L), jnp.float32),
            pltpu.VMEM((NT, 8, L), jnp.float32),
            pltpu.VMEM((L,), jnp.float32),
            pltpu.VMEM_SHARED((2 * N_ELEM,), jnp.float32),
            pltpu.VMEM_SHARED((2 * N_ELEM,), jnp.float32),
            pltpu.VMEM_SHARED((NT, 8, L), jnp.float32),
        ],
    )
    return f(logits)


def kernel(gt_logits, gen_logits):
    logits = jnp.concatenate(
        [jnp.squeeze(gt_logits, -1), jnp.squeeze(gen_logits, -1)], axis=0)
    out = _run(logits)
    (accuracy, precision, recall, f1, pr_auc, roc_auc,
     gap, prob_fake, ppf, acc05) = (out[i] for i in range(10))
    return (accuracy, precision, recall, f1, pr_auc, roc_auc,
            gap, prob_fake, ppf, acc05)


# speedup vs baseline: 2.1733x; 1.0067x over previous
import jax
import jax.numpy as jnp
from jax import lax
from jax.experimental import pallas as pl
from jax.experimental.pallas import tpu as pltpu
from jax.experimental.pallas import tpu_sc as plsc

N_ELEM = 16384
NT = 16
CH = 2048
NVC = CH // 16
QB = N_ELEM // NT
L = 16


def _log2(x):
    return x.bit_length() - 1


def _sigmoid(x):
    e = jnp.exp(-jnp.abs(x))
    return jnp.where(x >= 0, 1.0 / (1.0 + e), e / (1.0 + e))


UNROLL = 8


def _sort_pass(buf, n_vregs):
    u = UNROLL if n_vregs % UNROLL == 0 else 1

    def body(i, _):
        for j in range(u):
            x = buf[pl.ds((i * u + j) * L, L)]
            sk, _unused = plsc.sort_key_val(x, x)
            buf[pl.ds((i * u + j) * L, L)] = sk
        return 0

    lax.fori_loop(0, n_vregs // u, body, 0)


def _ce_stages(buf, size, s_hi, s_lo=16):
    s = s_hi
    n_units = size // (2 * L)
    while s >= s_lo:
        sv = s // L
        shift = _log2(sv)
        mask = sv - 1
        uu = UNROLL if n_units % UNROLL == 0 else 1

        def body(i, _, s=s, shift=shift, mask=mask, sv=sv, uu=uu):
            for j in range(uu):
                u = i * uu + j
                g = lax.shift_right_logical(u, shift)
                r = lax.bitwise_and(u, mask)
                b1 = (g * (2 * sv) + r) * L
                b2 = b1 + s
                a = buf[pl.ds(b1, L)]
                b = buf[pl.ds(b2, L)]
                buf[pl.ds(b1, L)] = jnp.minimum(a, b)
                buf[pl.ds(b2, L)] = jnp.maximum(a, b)
            return 0

        lax.fori_loop(0, n_units // uu, body, 0)
        s //= 2


def _fused_level(buf, size, m, presort=False):
    nb = 2 * m // L
    nhalf = nb // 2
    nblocks = size // (2 * m)
    bpi = max(1, 8 // nb)

    def body(i, _, nb=nb, nhalf=nhalf, bpi=bpi):
        for bb in range(bpi):
            base_v = (i * bpi + bb) * nb
            v = [buf[pl.ds((base_v + j) * L, L)] for j in range(nb)]
            if presort:
                for j in range(nb):
                    v[j], _unused = plsc.sort_key_val(v[j], v[j])
            for j in range(nhalf):
                bflip = jnp.flip(v[nb - 1 - j], 0)
                lo = jnp.minimum(v[j], bflip)
                hi = jnp.maximum(v[j], bflip)
                v[j] = lo
                v[nb - 1 - j] = jnp.flip(hi, 0)
            sv = nb // 4
            while sv >= 1:
                for g0 in range(0, nb, 2 * sv):
                    for j in range(sv):
                        a = v[g0 + j]
                        b = v[g0 + j + sv]
                        v[g0 + j] = jnp.minimum(a, b)
                        v[g0 + j + sv] = jnp.maximum(a, b)
                sv //= 2
            for j in range(nb):
                sk, _unused = plsc.sort_key_val(v[j], v[j])
                buf[pl.ds((base_v + j) * L, L)] = sk
        return 0

    lax.fori_loop(0, nblocks // bpi, body, 0)


def _mirror_pass(buf, size, m):
    nv = m // L
    shift = _log2(nv)
    mask = nv - 1
    n_units = size // (2 * L)
    uu = UNROLL if n_units % UNROLL == 0 else 1

    def body(i, _):
        for jj in range(uu):
            u = i * uu + jj
            p = lax.shift_right_logical(u, shift)
            j = lax.bitwise_and(u, mask)
            ai = (p * 2 * nv + j) * L
            bi = (p * 2 * nv + 2 * nv - 1 - j) * L
            a = buf[pl.ds(ai, L)]
            b = jnp.flip(buf[pl.ds(bi, L)], 0)
            buf[pl.ds(ai, L)] = jnp.minimum(a, b)
            buf[pl.ds(bi, L)] = jnp.flip(jnp.maximum(a, b), 0)
        return 0

    lax.fori_loop(0, n_units // uu, body, 0)


def _tail8(buf, size):
    def body(i, _):
        base_v = i * 8
        v = [buf[pl.ds((base_v + j) * L, L)] for j in range(8)]
        for sv in (4, 2, 1):
            for g0 in range(0, 8, 2 * sv):
                for j in range(sv):
                    a = v[g0 + j]
                    b = v[g0 + j + sv]
                    v[g0 + j] = jnp.minimum(a, b)
                    v[g0 + j + sv] = jnp.maximum(a, b)
        for j in range(8):
            sk, _unused = plsc.sort_key_val(v[j], v[j])
            buf[pl.ds((base_v + j) * L, L)] = sk
        return 0

    lax.fori_loop(0, size // (8 * L), body, 0)


def _local_sort(buf, size):
    m = 16
    while m < size:
        if m <= 128:
            _fused_level(buf, size, m, presort=(m == 16))
        else:
            _mirror_pass(buf, size, m)
            _ce_stages(buf, size, m // 2, s_lo=128)
            _tail8(buf, size)
        m *= 2


def _count_le(arr, q, strict=False):
    lo = jnp.zeros((L,), jnp.int32)
    s = N_ELEM // 2
    while s >= 1:
        idx = lo + (s - 1)
        v = plsc.load_gather(arr, [idx])
        take = (v < q) if strict else (v <= q)
        lo = jnp.where(take, lo + s, lo)
        s //= 2
    return lo


def _tec_body(logits_hbm, out_hbm,
              lg, keys_c, pkeys, sfall, stats_my, stats_all, outv,
              sh_a, sh_b, shared_stats):
    w = lax.axis_index("s")
    base = w * CH

    def cross_mirror(src, dst, k, finish=False):
        c = lax.bitwise_and(w, 7)
        g = lax.bitwise_and(c, 2 * k - 1)
        part_l = (c - g) + (2 * k - 1 - g)
        part = lax.bitwise_or(lax.bitwise_and(w, 8), part_l)
        is_min = jnp.broadcast_to(g < k, (L,))
        pltpu.sync_copy(src.at[pl.ds(base, CH)], keys_c)
        pltpu.sync_copy(src.at[pl.ds(part * CH, CH)], pkeys)

        def body(i, _):
            for jj in range(UNROLL):
                j = i * UNROLL + jj
                a = keys_c[pl.ds(j * L, L)]
                b = jnp.flip(pkeys[pl.ds((NVC - 1) * L - j * L, L)], 0)
                keys_c[pl.ds(j * L, L)] = jnp.where(
                    is_min, jnp.minimum(a, b), jnp.maximum(a, b))
            return 0

        lax.fori_loop(0, NVC // UNROLL, body, 0)
        if finish:
            _ce_stages(keys_c, CH, CH // 2, s_lo=128)
            _tail8(keys_c, CH)
        pltpu.sync_copy(keys_c, dst.at[pl.ds(base, CH)])
        plsc.subcore_barrier()

    def cross_aligned(src, dst, sc, finish=False):
        part = lax.bitwise_xor(w, sc)
        is_min = jnp.broadcast_to(lax.bitwise_and(w, sc) == 0, (L,))
        pltpu.sync_copy(src.at[pl.ds(base, CH)], keys_c)
        pltpu.sync_copy(src.at[pl.ds(part * CH, CH)], pkeys)

        def body(i, _):
            for jj in range(UNROLL):
                j = i * UNROLL + jj
                a = keys_c[pl.ds(j * L, L)]
                b = pkeys[pl.ds(j * L, L)]
                keys_c[pl.ds(j * L, L)] = jnp.where(
                    is_min, jnp.minimum(a, b), jnp.maximum(a, b))
            return 0

        lax.fori_loop(0, NVC // UNROLL, body, 0)
        if finish:
            _ce_stages(keys_c, CH, CH // 2, s_lo=128)
            _tail8(keys_c, CH)
        pltpu.sync_copy(keys_c, dst.at[pl.ds(base, CH)])
        plsc.subcore_barrier()

    pltpu.sync_copy(logits_hbm.at[pl.ds(base, CH)], lg)
    zero = jnp.zeros((L,), jnp.float32)
    for j in range(8):
        stats_my[j] = zero

    def sig_body(i, _):
        acc = jnp.zeros((L,), jnp.float32)
        for jj in range(UNROLL):
            j = i * UNROLL + jj
            x = lg[pl.ds(j * L, L)]
            sv = _sigmoid(x)
            acc = acc + sv
            keys_c[pl.ds(j * L, L)] = sv
        stats_my[2] = stats_my[2] + acc
        return 0

    lax.fori_loop(0, NVC // UNROLL, sig_body, 0)

    _local_sort(keys_c, CH)
    pltpu.sync_copy(keys_c, sh_a.at[pl.ds(base, CH)])
    plsc.subcore_barrier()

    cross_mirror(sh_a, sh_b, 1, finish=True)
    cross_mirror(sh_b, sh_a, 2)
    cross_aligned(sh_a, sh_b, 1, finish=True)
    cross_mirror(sh_b, sh_a, 4)
    cross_aligned(sh_a, sh_b, 2)
    cross_aligned(sh_b, sh_a, 1, finish=True)

    pltpu.sync_copy(sh_a.at[pl.ds(N_ELEM, N_ELEM)], sfall.at[pl.ds(0, N_ELEM)])
    pltpu.sync_copy(sh_a.at[pl.ds(w * QB, QB)], lg.at[pl.ds(0, QB)])
    iota = lax.iota(jnp.int32, L)

    def search_body(i, _):
        pr_t = jnp.zeros((L,), jnp.float32)
        rr_t = jnp.zeros((L,), jnp.float32)
        for jj in range(4):
            v = i * 4 + jj
            q = lg[pl.ds(v * L, L)]
            pos = iota + (w * QB + v * L)
            cnt_le = _count_le(sfall, q)
            c_gt = (N_ELEM - cnt_le).astype(jnp.float32)
            r = (N_ELEM - pos).astype(jnp.float32)
            big_r = r + c_gt
            pr_t = pr_t + r / (big_r + 1e-08)
            rr_t = rr_t + big_r
        stats_my[3] = stats_my[3] + pr_t
        stats_my[4] = stats_my[4] + rr_t
        return 0

    lax.fori_loop(0, QB // L // 4, search_body, 0)

    pltpu.sync_copy(stats_my, shared_stats.at[w])
    plsc.subcore_barrier()

    @pl.when(w == 0)
    def _():
        q05 = jnp.full((L,), 0.5, jnp.float32)
        le_f = _count_le(sfall, q05)
        lt_f = _count_le(sfall, q05, strict=True)
        pltpu.sync_copy(sh_a.at[pl.ds(0, N_ELEM)], sfall.at[pl.ds(0, N_ELEM)])
        le_r = _count_le(sfall, q05)
        lt_r = _count_le(sfall, q05, strict=True)
        n_pr = (N_ELEM - le_r).astype(jnp.float32)
        n_pf = (N_ELEM - le_f).astype(jnp.float32)
        eq_r = (le_r - lt_r).astype(jnp.float32)
        eq_f = (le_f - lt_f).astype(jnp.float32)

        pltpu.sync_copy(shared_stats, stats_all)

        def red(row, tiles):
            acc = jnp.zeros((L,), jnp.float32)
            for t in tiles:
                acc = acc + stats_all[t, row]
            return jnp.broadcast_to(jnp.sum(acc), (L,))

        reals = range(8)
        fakes = range(8, NT)
        sum_r = red(2, reals)
        sum_f = red(2, fakes)
        pr_sum = red(3, range(NT))
        r_sum = red(4, range(NT))

        p = jnp.full((L,), float(N_ELEM), jnp.float32)
        n = jnp.full((L,), float(N_ELEM), jnp.float32)
        accuracy = (n_pr + (n - n_pf)) / (p + n)
        tp = n_pr
        fp = n_pf
        fn = p - n_pr
        precision = tp / (tp + fp + 1e-08)
        recall = tp / (tp + fn + 1e-08)
        f1 = 2.0 * precision * recall / (precision + recall + 1e-08)
        pr_auc = pr_sum / (p + 1e-08)
        sum_pos_ranks = float((2 * N_ELEM + 1) * N_ELEM) - r_sum
        roc_auc = (sum_pos_ranks - float(N_ELEM * (N_ELEM + 1) // 2)) \
            / float(N_ELEM * N_ELEM)
        gap = sum_r / p - sum_f / n
        prob_fake = sum_f / n
        ppf = (n_pr + n_pf) / (p + n)
        acc05 = (eq_r / p + eq_f / n) * 0.5

        vals = [accuracy, precision, recall, f1, pr_auc, roc_auc,
                gap, prob_fake, ppf, acc05]
        ov = jnp.zeros((L,), jnp.float32)
        for i, s in enumerate(vals):
            ov = jnp.where(iota == i, s, ov)
        outv[...] = ov
        pltpu.sync_copy(outv, out_hbm)


@jax.jit
def _run(logits):
    mesh = plsc.VectorSubcoreMesh(
        core_axis_name="c", subcore_axis_name="s", num_cores=1)
    f = pl.kernel(
        _tec_body,
        out_type=jax.ShapeDtypeStruct((L,), jnp.float32),
        mesh=mesh,
        compiler_params=pltpu.CompilerParams(needs_layout_passes=False),
        scratch_types=[
            pltpu.VMEM((CH,), jnp.float32),
            pltpu.VMEM((CH,), jnp.float32),
            pltpu.VMEM((CH,), jnp.float32),
            pltpu.VMEM((N_ELEM + 256,), jnp.float32),
            pltpu.VMEM((8, L), jnp.float32),
            pltpu.VMEM((NT, 8, L), jnp.float32),
            pltpu.VMEM((L,), jnp.float32),
            pltpu.VMEM_SHARED((2 * N_ELEM,), jnp.float32),
            pltpu.VMEM_SHARED((2 * N_ELEM,), jnp.float32),
            pltpu.VMEM_SHARED((NT, 8, L), jnp.float32),
        ],
    )
    return f(logits)


def kernel(gt_logits, gen_logits):
    logits = jnp.concatenate(
        [jnp.squeeze(gt_logits, -1), jnp.squeeze(gen_logits, -1)], axis=0)
    out = _run(logits)
    (accuracy, precision, recall, f1, pr_auc, roc_auc,
     gap, prob_fake, ppf, acc05) = (out[i] for i in range(10))
    return (accuracy, precision, recall, f1, pr_auc, roc_auc,
            gap, prob_fake, ppf, acc05)
